# Initial kernel scaffold; baseline (speedup 1.0000x reference)
import jax
import jax.numpy as jnp
from jax.experimental import pallas as pl


def kernel(x, edge_index, edge_attr, We0, be0, W10, b10, W20, b20, eps0, g0, bt0, We1, be1, W11, b11, W21, b21, eps1, g1, bt1):
    raise NotImplementedError("write your pallas kernel here")


# speedup vs baseline: 1.8855x; 1.8855x over previous
import functools

import jax
import jax.numpy as jnp
from jax import lax
from jax.experimental import pallas as pl
from jax.experimental.pallas import tpu as pltpu
from jax.experimental.pallas import tpu_sc as plsc

N = 10000
E = 320000
D = 128
DE = 16

NC = 2
NS = 16
NW = NC * NS
EPW = E // NW
G = 80
KCH = EPW // G
CHO = 200
NCH = N // CHO


def _edge_linear_body(ea_ref, we0_ref, be0_ref, we1_ref, be1_ref,
                      ee0_ref, ee1_ref):
    ea = ea_ref[...]
    ee0_ref[...] = jnp.dot(ea, we0_ref[...],
                           preferred_element_type=jnp.float32) + be0_ref[...]
    ee1_ref[...] = jnp.dot(ea, we1_ref[...],
                           preferred_element_type=jnp.float32) + be1_ref[...]


def _edge_linear(edge_attr, We0, be0, We1, be1):
    BE = 6400
    grid = (E // BE,)
    return pl.pallas_call(
        _edge_linear_body,
        grid=grid,
        in_specs=[
            pl.BlockSpec((BE, DE), lambda i: (i, 0)),
            pl.BlockSpec((DE, D), lambda i: (0, 0)),
            pl.BlockSpec((1, D), lambda i: (0, 0)),
            pl.BlockSpec((DE, D), lambda i: (0, 0)),
            pl.BlockSpec((1, D), lambda i: (0, 0)),
        ],
        out_specs=[
            pl.BlockSpec((BE, D), lambda i: (i, 0)),
            pl.BlockSpec((BE, D), lambda i: (i, 0)),
        ],
        out_shape=[
            jax.ShapeDtypeStruct((E, D), jnp.float32),
            jax.ShapeDtypeStruct((E, D), jnp.float32),
        ],
    )(edge_attr, We0, be0.reshape(1, D), We1, be1.reshape(1, D))


def _edge_stage_body(x_hbm, ee_hbm, src_hbm, dst_hbm, out_hbm,
                     src_idx, dst_idx, rows, eebuf, zbuf, agg_sh, sem):
    cid = lax.axis_index("c")
    sid = lax.axis_index("s")
    wid = sid * NC + cid
    base_w = wid * EPW

    zv = jnp.zeros((16,), jnp.float32)

    @pl.loop(0, CHO * D // 16)
    def _zero_fill(j):
        r = j // (D // 16)
        c = (j % (D // 16)) * 16
        zbuf[r, pl.ds(c, 16)] = zv

    @pl.loop(sid, NCH, step=NS)
    def _zero_out(j):
        pltpu.sync_copy(zbuf, agg_sh.at[pl.ds(j * CHO, CHO)])

    plsc.subcore_barrier()

    @pl.loop(0, KCH)
    def _chunk(k):
        base = base_w + k * G
        pltpu.sync_copy(src_hbm.at[pl.ds(base, G)], src_idx)
        pltpu.sync_copy(dst_hbm.at[pl.ds(base, G)], dst_idx)
        gather = pltpu.async_copy(x_hbm.at[src_idx], rows, sem)
        pltpu.sync_copy(ee_hbm.at[pl.ds(base, G), :], eebuf)
        gather.wait()

        @pl.loop(0, G * D // 16)
        def _relu_add(j):
            r = j // (D // 16)
            c = (j % (D // 16)) * 16
            v = rows[r, pl.ds(c, 16)] + eebuf[r, pl.ds(c, 16)]
            rows[r, pl.ds(c, 16)] = jnp.maximum(v, 0.0)

        pltpu.sync_copy(rows, agg_sh.at[dst_idx], add=True)

    plsc.subcore_barrier()

    @pl.loop(sid, NCH, step=NS)
    def _copy_out(j):
        r0 = j * CHO
        pltpu.sync_copy(agg_sh.at[pl.ds(r0, CHO)],
                        out_hbm.at[cid, pl.ds(r0, CHO), :])


def _edge_stage(x, ee, src, dst):
    mesh = plsc.VectorSubcoreMesh(core_axis_name="c", subcore_axis_name="s",
                                  num_cores=NC, num_subcores=NS)
    f = pl.kernel(
        _edge_stage_body,
        out_type=jax.ShapeDtypeStruct((NC, N, D), jnp.float32),
        mesh=mesh,
        scratch_types=[
            pltpu.VMEM((G,), jnp.int32),
            pltpu.VMEM((G,), jnp.int32),
            pltpu.VMEM((G, D), jnp.float32),
            pltpu.VMEM((G, D), jnp.float32),
            pltpu.VMEM((CHO, D), jnp.float32),
            pltpu.VMEM_SHARED((N, D), jnp.float32),
            pltpu.SemaphoreType.DMA,
        ],
    )
    return f(x, ee, src, dst)


def _node_stage_body(x_ref, agg_ref, eps_ref, w1_ref, b1_ref, w2_ref, b2_ref,
                     g_ref, bt_ref, out_ref):
    x = x_ref[...]
    h = (1.0 + eps_ref[0, 0]) * x + agg_ref[0] + agg_ref[1]
    t = jnp.maximum(jnp.dot(h, w1_ref[...],
                            preferred_element_type=jnp.float32) + b1_ref[...],
                    0.0)
    o = jnp.dot(t, w2_ref[...],
                preferred_element_type=jnp.float32) + b2_ref[...]
    mu = jnp.mean(o, axis=-1, keepdims=True)
    var = jnp.mean(jnp.square(o - mu), axis=-1, keepdims=True)
    o = (o - mu) * lax.rsqrt(var + 1e-5) * g_ref[...] + bt_ref[...]
    out_ref[...] = jnp.maximum(o, 0.0)


def _node_stage(x, agg, eps, W1, b1, W2, b2, g, bt):
    BN = 2000
    grid = (N // BN,)
    return pl.pallas_call(
        _node_stage_body,
        grid=grid,
        in_specs=[
            pl.BlockSpec((BN, D), lambda i: (i, 0)),
            pl.BlockSpec((NC, BN, D), lambda i: (0, i, 0)),
            pl.BlockSpec((1, 1), lambda i: (0, 0)),
            pl.BlockSpec((D, D), lambda i: (0, 0)),
            pl.BlockSpec((1, D), lambda i: (0, 0)),
            pl.BlockSpec((D, D), lambda i: (0, 0)),
            pl.BlockSpec((1, D), lambda i: (0, 0)),
            pl.BlockSpec((1, D), lambda i: (0, 0)),
            pl.BlockSpec((1, D), lambda i: (0, 0)),
        ],
        out_specs=pl.BlockSpec((BN, D), lambda i: (i, 0)),
        out_shape=jax.ShapeDtypeStruct((N, D), jnp.float32),
    )(x, agg, eps.reshape(1, 1), W1, b1.reshape(1, D), W2, b2.reshape(1, D),
      g.reshape(1, D), bt.reshape(1, D))


def kernel(x, edge_index, edge_attr,
           We0, be0, W10, b10, W20, b20, eps0, g0, bt0,
           We1, be1, W11, b11, W21, b21, eps1, g1, bt1):
    src = edge_index[0].astype(jnp.int32)
    dst = edge_index[1].astype(jnp.int32)

    ee0, ee1 = _edge_linear(edge_attr, We0, be0, We1, be1)

    agg0 = _edge_stage(x, ee0, src, dst)
    h1 = _node_stage(x, agg0, eps0, W10, b10, W20, b20, g0, bt0)

    agg1 = _edge_stage(h1, ee1, src, dst)
    h2 = _node_stage(h1, agg1, eps1, W11, b11, W21, b21, g1, bt1)
    return h2


# speedup vs baseline: 5.1132x; 2.7118x over previous
import functools

import jax
import jax.numpy as jnp
from jax import lax
from jax.experimental import pallas as pl
from jax.experimental.pallas import tpu as pltpu
from jax.experimental.pallas import tpu_sc as plsc

N = 10000
E = 320000
D = 128
DE = 16

NC = 2
NS = 16
NW = NC * NS
EPW = E // NW
G = 80
KCH = EPW // G
CHO = 80
NCH = N // CHO


def _edge_linear_body(ea_ref, we0_ref, be0_ref, we1_ref, be1_ref,
                      ee0_ref, ee1_ref):
    ea = ea_ref[...]
    ee0_ref[...] = jnp.dot(ea, we0_ref[...],
                           preferred_element_type=jnp.float32) + be0_ref[...]
    ee1_ref[...] = jnp.dot(ea, we1_ref[...],
                           preferred_element_type=jnp.float32) + be1_ref[...]


def _edge_linear(edge_attr, We0, be0, We1, be1):
    BE = 6400
    grid = (E // BE,)
    return pl.pallas_call(
        _edge_linear_body,
        grid=grid,
        in_specs=[
            pl.BlockSpec((BE, DE), lambda i: (i, 0)),
            pl.BlockSpec((DE, D), lambda i: (0, 0)),
            pl.BlockSpec((1, D), lambda i: (0, 0)),
            pl.BlockSpec((DE, D), lambda i: (0, 0)),
            pl.BlockSpec((1, D), lambda i: (0, 0)),
        ],
        out_specs=[
            pl.BlockSpec((BE, D), lambda i: (i, 0)),
            pl.BlockSpec((BE, D), lambda i: (i, 0)),
        ],
        out_shape=[
            jax.ShapeDtypeStruct((E, D), jnp.float32),
            jax.ShapeDtypeStruct((E, D), jnp.float32),
        ],
    )(edge_attr, We0, be0.reshape(1, D), We1, be1.reshape(1, D))


def _edge_stage_body(x_hbm, ee_hbm, src_hbm, dst_hbm, out_hbm,
                     src_idx, dst_idx, rows, eebuf, agg_sh,
                     sg0, sg1, se0, se1, ss0, ss1, si0, si1, si2, si3):
    cid = lax.axis_index("c")
    sid = lax.axis_index("s")
    wid = sid * NC + cid
    base_w = wid * EPW

    sem_g = [sg0, sg1]
    sem_e = [se0, se1]
    sem_s = [ss0, ss1]
    sem_i = [si0, si1, si2, si3]

    zv = jnp.zeros((16,), jnp.float32)

    @pl.loop(0, G)
    def _zero_fill(r):
        for cc in range(D // 16):
            rows[0, r, pl.ds(cc * 16, 16)] = zv

    @pl.loop(sid, NCH, step=NS)
    def _zero_out(j):
        pltpu.sync_copy(rows.at[0], agg_sh.at[pl.ds(j * CHO, CHO)])

    plsc.subcore_barrier()

    def issue_idx(k, islot):
        base = base_w + k * G
        pltpu.async_copy(src_hbm.at[pl.ds(base, G)], src_idx.at[islot],
                         sem_i[islot])
        pltpu.async_copy(dst_hbm.at[pl.ds(base, G)], dst_idx.at[islot],
                         sem_i[islot])

    def wait_idx(islot):
        pltpu.make_async_copy(src_hbm.at[pl.ds(0, G)], src_idx.at[islot],
                              sem_i[islot]).wait()
        pltpu.make_async_copy(dst_hbm.at[pl.ds(0, G)], dst_idx.at[islot],
                              sem_i[islot]).wait()

    def issue_gather_ee(k, b, islot):
        base = base_w + k * G
        pltpu.async_copy(x_hbm.at[src_idx.at[islot]], rows.at[b], sem_g[b])
        pltpu.async_copy(ee_hbm.at[pl.ds(base, G), :], eebuf.at[b], sem_e[b])

    def phase(k, p):
        b = p % 2
        b1 = (p + 1) % 2
        pltpu.make_async_copy(x_hbm.at[pl.ds(0, G), :], rows.at[b],
                              sem_g[b]).wait()
        pltpu.make_async_copy(ee_hbm.at[pl.ds(0, G), :], eebuf.at[b],
                              sem_e[b]).wait()

        @pl.when(k + 1 < KCH)
        def _():
            @pl.when(k >= 1)
            def _():
                pltpu.make_async_copy(rows.at[b1], agg_sh.at[pl.ds(0, G)],
                                      sem_s[b1]).wait()
            wait_idx((p + 1) % 4)
            issue_gather_ee(k + 1, b1, (p + 1) % 4)

        @pl.when(k + 2 < KCH)
        def _():
            issue_idx(k + 2, (p + 2) % 4)

        @pl.loop(0, G)
        def _relu_add(r):
            for cc in range(D // 16):
                c = cc * 16
                v = rows[b, r, pl.ds(c, 16)] + eebuf[b, r, pl.ds(c, 16)]
                rows[b, r, pl.ds(c, 16)] = jnp.maximum(v, 0.0)

        pltpu.async_copy(rows.at[b], agg_sh.at[dst_idx.at[p % 4]], sem_s[b],
                         add=True)

    issue_idx(0, 0)
    issue_idx(1, 1)
    wait_idx(0)
    issue_gather_ee(0, 0, 0)

    @pl.loop(0, KCH - 1, step=4)
    def _main(k0):
        for p in range(4):
            phase(k0 + p, p)

    phase(KCH - 1, (KCH - 1) % 4)

    pltpu.make_async_copy(rows.at[0], agg_sh.at[pl.ds(0, G)], sem_s[0]).wait()
    pltpu.make_async_copy(rows.at[1], agg_sh.at[pl.ds(0, G)], sem_s[1]).wait()

    plsc.subcore_barrier()

    @pl.loop(sid, NCH, step=NS)
    def _copy_out(j):
        r0 = j * CHO
        pltpu.sync_copy(agg_sh.at[pl.ds(r0, CHO)],
                        out_hbm.at[cid, pl.ds(r0, CHO), :])


def _edge_stage(x, ee, src, dst):
    mesh = plsc.VectorSubcoreMesh(core_axis_name="c", subcore_axis_name="s",
                                  num_cores=NC, num_subcores=NS)
    f = pl.kernel(
        _edge_stage_body,
        out_type=jax.ShapeDtypeStruct((NC, N, D), jnp.float32),
        mesh=mesh,
        scratch_types=[
            pltpu.VMEM((4, G), jnp.int32),
            pltpu.VMEM((4, G), jnp.int32),
            pltpu.VMEM((2, G, D), jnp.float32),
            pltpu.VMEM((2, G, D), jnp.float32),
            pltpu.VMEM_SHARED((N, D), jnp.float32),
        ] + [pltpu.SemaphoreType.DMA] * 10,
    )
    return f(x, ee, src, dst)


def _node_stage_body(x_ref, agg_ref, eps_ref, w1_ref, b1_ref, w2_ref, b2_ref,
                     g_ref, bt_ref, out_ref):
    x = x_ref[...]
    h = (1.0 + eps_ref[0, 0]) * x + agg_ref[0] + agg_ref[1]
    t = jnp.maximum(jnp.dot(h, w1_ref[...],
                            preferred_element_type=jnp.float32) + b1_ref[...],
                    0.0)
    o = jnp.dot(t, w2_ref[...],
                preferred_element_type=jnp.float32) + b2_ref[...]
    mu = jnp.mean(o, axis=-1, keepdims=True)
    var = jnp.mean(jnp.square(o - mu), axis=-1, keepdims=True)
    o = (o - mu) * lax.rsqrt(var + 1e-5) * g_ref[...] + bt_ref[...]
    out_ref[...] = jnp.maximum(o, 0.0)


def _node_stage(x, agg, eps, W1, b1, W2, b2, g, bt):
    BN = 2000
    grid = (N // BN,)
    return pl.pallas_call(
        _node_stage_body,
        grid=grid,
        in_specs=[
            pl.BlockSpec((BN, D), lambda i: (i, 0)),
            pl.BlockSpec((NC, BN, D), lambda i: (0, i, 0)),
            pl.BlockSpec((1, 1), lambda i: (0, 0)),
            pl.BlockSpec((D, D), lambda i: (0, 0)),
            pl.BlockSpec((1, D), lambda i: (0, 0)),
            pl.BlockSpec((D, D), lambda i: (0, 0)),
            pl.BlockSpec((1, D), lambda i: (0, 0)),
            pl.BlockSpec((1, D), lambda i: (0, 0)),
            pl.BlockSpec((1, D), lambda i: (0, 0)),
        ],
        out_specs=pl.BlockSpec((BN, D), lambda i: (i, 0)),
        out_shape=jax.ShapeDtypeStruct((N, D), jnp.float32),
    )(x, agg, eps.reshape(1, 1), W1, b1.reshape(1, D), W2, b2.reshape(1, D),
      g.reshape(1, D), bt.reshape(1, D))


def kernel(x, edge_index, edge_attr,
           We0, be0, W10, b10, W20, b20, eps0, g0, bt0,
           We1, be1, W11, b11, W21, b21, eps1, g1, bt1):
    src = edge_index[0].astype(jnp.int32)
    dst = edge_index[1].astype(jnp.int32)

    ee0, ee1 = _edge_linear(edge_attr, We0, be0, We1, be1)

    agg0 = _edge_stage(x, ee0, src, dst)
    h1 = _node_stage(x, agg0, eps0, W10, b10, W20, b20, g0, bt0)

    agg1 = _edge_stage(h1, ee1, src, dst)
    h2 = _node_stage(h1, agg1, eps1, W11, b11, W21, b21, g1, bt1)
    return h2
